# Initial kernel scaffold; baseline (speedup 1.0000x reference)
import jax
import jax.numpy as jnp
from jax.experimental import pallas as pl


def kernel(x, embed):
    raise NotImplementedError("write your pallas kernel here")


# speedup vs baseline: 1.6454x; 1.6454x over previous
import jax
import jax.numpy as jnp
from jax.experimental import pallas as pl

N_ROWS = 64 * 576
E_DIM = 64
N_CODES = 1024
BLOCK_R = 1024


def _vq_kernel(x_ref, embed_ref, quant_ref, idx_ref):
    xb = x_ref[...]
    emb = embed_ref[...]
    inv_sq = 1.0 / jnp.sum(emb * emb, axis=1, keepdims=True)
    scaled = emb * inv_sq
    sim = jax.lax.dot_general(
        xb, scaled, (((1,), (1,)), ((), ())),
        preferred_element_type=jnp.float32)
    idx = jnp.argmax(sim, axis=1).astype(jnp.int32)
    onehot = (jax.lax.broadcasted_iota(jnp.int32, sim.shape, 1)
              == idx[:, None]).astype(jnp.float32)
    quant = jax.lax.dot_general(
        onehot, emb, (((1,), (0,)), ((), ())),
        preferred_element_type=jnp.float32)
    quant_ref[...] = quant
    idx_ref[...] = idx


def kernel(x, embed):
    flat = x.reshape(N_ROWS, E_DIM)
    grid = (N_ROWS // BLOCK_R,)
    quant, idx = pl.pallas_call(
        _vq_kernel,
        grid=grid,
        in_specs=[
            pl.BlockSpec((BLOCK_R, E_DIM), lambda i: (i, 0)),
            pl.BlockSpec((N_CODES, E_DIM), lambda i: (0, 0)),
        ],
        out_specs=[
            pl.BlockSpec((BLOCK_R, E_DIM), lambda i: (i, 0)),
            pl.BlockSpec((BLOCK_R,), lambda i: (i,)),
        ],
        out_shape=[
            jax.ShapeDtypeStruct((N_ROWS, E_DIM), jnp.float32),
            jax.ShapeDtypeStruct((N_ROWS,), jnp.int32),
        ],
    )(flat, embed)
    return quant, idx


# speedup vs baseline: 1.8490x; 1.1238x over previous
import functools
import jax
import jax.numpy as jnp
from jax import lax
from jax.experimental import pallas as pl
from jax.experimental.pallas import tpu as pltpu
from jax.experimental.pallas import tpu_sc as plsc

N_ROWS = 64 * 576
E_DIM = 64
N_CODES = 1024
BLOCK_R = 1024

NW = 32
B_PER_W = N_ROWS // NW
CHUNK = 128
N_CHUNKS = B_PER_W // CHUNK


def _argmax_kernel(x_ref, embed_ref, idx_ref):
    xb = x_ref[...]
    emb = embed_ref[...]
    inv_sq = 1.0 / jnp.sum(emb * emb, axis=1, keepdims=True)
    scaled = emb * inv_sq
    sim = jax.lax.dot_general(
        xb, scaled, (((1,), (1,)), ((), ())),
        preferred_element_type=jnp.float32)
    idx_ref[...] = jnp.argmax(sim, axis=1).astype(jnp.int32)


def _tc_argmax(flat, embed):
    grid = (N_ROWS // BLOCK_R,)
    return pl.pallas_call(
        _argmax_kernel,
        grid=grid,
        in_specs=[
            pl.BlockSpec((BLOCK_R, E_DIM), lambda i: (i, 0)),
            pl.BlockSpec((N_CODES, E_DIM), lambda i: (0, 0)),
        ],
        out_specs=pl.BlockSpec((BLOCK_R,), lambda i: (i,)),
        out_shape=jax.ShapeDtypeStruct((N_ROWS,), jnp.int32),
    )(flat, embed)


_sc_mesh = plsc.VectorSubcoreMesh(core_axis_name="c", subcore_axis_name="s")


@functools.partial(
    pl.kernel,
    mesh=_sc_mesh,
    compiler_params=pltpu.CompilerParams(use_tc_tiling_on_sc=False),
    out_type=jax.ShapeDtypeStruct((N_ROWS, E_DIM), jnp.float32),
    scratch_types=[
        pltpu.VMEM((B_PER_W,), jnp.int32),
        pltpu.VMEM((B_PER_W, E_DIM), jnp.float32),
        pltpu.SemaphoreType.DMA,
    ],
)
def _sc_gather(idx_hbm, embed_hbm, out_hbm, idx_v, rows_v, sem):
    wid = lax.axis_index("s") * 2 + lax.axis_index("c")
    base = wid * B_PER_W
    pltpu.sync_copy(idx_hbm.at[pl.ds(base, B_PER_W)], idx_v)
    copies = []
    for j in range(N_CHUNKS):
        copies.append(pltpu.async_copy(
            embed_hbm.at[idx_v.at[pl.ds(j * CHUNK, CHUNK)]],
            rows_v.at[pl.ds(j * CHUNK, CHUNK)],
            sem))
    for c in copies:
        c.wait()
    pltpu.sync_copy(rows_v, out_hbm.at[pl.ds(base, B_PER_W)])


def kernel(x, embed):
    flat = x.reshape(N_ROWS, E_DIM)
    idx = _tc_argmax(flat, embed)
    quant = _sc_gather(idx, embed)
    return quant, idx


# speedup vs baseline: 2.4323x; 1.3154x over previous
import functools
import jax
import jax.numpy as jnp
from jax import lax
from jax.experimental import pallas as pl
from jax.experimental.pallas import tpu as pltpu
from jax.experimental.pallas import tpu_sc as plsc

N_ROWS = 64 * 576
E_DIM = 64
N_CODES = 1024
BLOCK_R = 1024

NW = 32
B_PER_W = N_ROWS // NW
CHUNK = 128
N_CHUNKS = B_PER_W // CHUNK


def _argmax_kernel(x_ref, embed_ref, idx_ref):
    xb = x_ref[...]
    emb = embed_ref[...]
    inv_sq = 1.0 / jnp.sum(emb * emb, axis=1, keepdims=True)
    scaled = emb * inv_sq
    simT = jax.lax.dot_general(
        scaled, xb, (((1,), (1,)), ((), ())),
        preferred_element_type=jnp.float32)
    idx_ref[...] = jnp.argmax(simT, axis=0).astype(jnp.int32)


def _tc_argmax(flat, embed):
    grid = (N_ROWS // BLOCK_R,)
    return pl.pallas_call(
        _argmax_kernel,
        grid=grid,
        in_specs=[
            pl.BlockSpec((BLOCK_R, E_DIM), lambda i: (i, 0)),
            pl.BlockSpec((N_CODES, E_DIM), lambda i: (0, 0)),
        ],
        out_specs=pl.BlockSpec((BLOCK_R,), lambda i: (i,)),
        out_shape=jax.ShapeDtypeStruct((N_ROWS,), jnp.int32),
    )(flat, embed)


_sc_mesh = plsc.VectorSubcoreMesh(core_axis_name="c", subcore_axis_name="s")


@functools.partial(
    pl.kernel,
    mesh=_sc_mesh,
    compiler_params=pltpu.CompilerParams(use_tc_tiling_on_sc=False),
    out_type=jax.ShapeDtypeStruct((N_ROWS, E_DIM), jnp.float32),
    scratch_types=[
        pltpu.VMEM((B_PER_W,), jnp.int32),
        pltpu.VMEM((B_PER_W, E_DIM), jnp.float32),
        pltpu.SemaphoreType.DMA,
    ],
)
def _sc_gather(idx_hbm, embed_hbm, out_hbm, idx_v, rows_v, sem):
    wid = lax.axis_index("s") * 2 + lax.axis_index("c")
    base = wid * B_PER_W
    pltpu.sync_copy(idx_hbm.at[pl.ds(base, B_PER_W)], idx_v)
    copies = []
    for j in range(N_CHUNKS):
        copies.append(pltpu.async_copy(
            embed_hbm.at[idx_v.at[pl.ds(j * CHUNK, CHUNK)]],
            rows_v.at[pl.ds(j * CHUNK, CHUNK)],
            sem))
    for c in copies:
        c.wait()
    pltpu.sync_copy(rows_v, out_hbm.at[pl.ds(base, B_PER_W)])


def kernel(x, embed):
    flat = x.reshape(N_ROWS, E_DIM)
    idx = _tc_argmax(flat, embed)
    quant = _sc_gather(idx, embed)
    return quant, idx
